# Initial kernel scaffold; baseline (speedup 1.0000x reference)
import jax
import jax.numpy as jnp
from jax.experimental import pallas as pl


def kernel(predicted, target, epoch):
    raise NotImplementedError("write your pallas kernel here")


# speedup vs baseline: 8.1551x; 8.1551x over previous
import jax
import jax.numpy as jnp
from jax.experimental import pallas as pl

_C = 64
_D = 128
_BLK = 3200


def _tc_body(tgt_ref, x_ref, sum_ref, cnt_ref):
    i = pl.program_id(0)

    @pl.when(i == 0)
    def _():
        sum_ref[...] = jnp.zeros_like(sum_ref)
        cnt_ref[...] = jnp.zeros_like(cnt_ref)

    x = x_ref[...]
    t = tgt_ref[0, 0, :]
    classes = jax.lax.broadcasted_iota(jnp.int32, (1, _C), 1)
    oh = (t[:, None] == classes).astype(jnp.float32)
    sum_ref[...] += jax.lax.dot_general(
        oh, x, (((0,), (0,)), ((), ())),
        preferred_element_type=jnp.float32)
    cnt_ref[...] += jnp.sum(oh, axis=0, keepdims=True)


def kernel(predicted, target, epoch):
    n, d = predicted.shape
    nb = n // _BLK
    tgt3 = target.reshape(nb, 1, _BLK)
    seg_sum, cnt = pl.pallas_call(
        _tc_body,
        grid=(nb,),
        in_specs=[
            pl.BlockSpec((1, 1, _BLK), lambda i: (i, 0, 0)),
            pl.BlockSpec((_BLK, _D), lambda i: (i, 0)),
        ],
        out_specs=[
            pl.BlockSpec((_C, _D), lambda i: (0, 0)),
            pl.BlockSpec((1, _C), lambda i: (0, 0)),
        ],
        out_shape=[
            jax.ShapeDtypeStruct((_C, _D), jnp.float32),
            jax.ShapeDtypeStruct((1, _C), jnp.float32),
        ],
    )(tgt3, predicted)
    cond = (epoch % 3) == 0
    seg_sum = jnp.where(cond, seg_sum, 0.0)
    count = jnp.where(cond, cnt.reshape(_C, 1), 0.0)
    loss = jnp.zeros((), jnp.float32)
    return (loss, seg_sum, count)
